# baseline (speedup 1.0000x reference)
import jax
import jax.numpy as jnp
from jax import lax
from jax.experimental import pallas as pl
from jax.experimental.pallas import tpu as pltpu
from jax.experimental.pallas import tpu_sc as plsc

B, S, D = 16, 2048, 1024
L = 16
HALF = D // 2


def _body(emb_hbm, mask_hbm, out_hbm, mask_v, d_v, sem0, sem1):
    w = lax.axis_index("s") * 2 + lax.axis_index("c")
    b = w // 2
    h = w % 2
    c0 = h * HALF

    cpe = pltpu.async_copy(
        emb_hbm.at[b, pl.ds(0, 2), pl.ds(c0, HALF)], d_v, sem1)
    cpm = pltpu.async_copy(mask_hbm, mask_v, sem0)
    cpm.wait()
    cpe.wait()

    iota = lax.iota(jnp.int32, L)
    bvec = ((iota + b) & (L - 1)).reshape(L, 1)
    dnums = lax.GatherDimensionNumbers(
        offset_dims=(), collapsed_slice_dims=(0,), start_index_map=(0,))
    m0b = lax.gather(mask_v[pl.ds(0, L)], bvec, dnums, (1,),
                     mode=lax.GatherScatterMode.PROMISE_IN_BOUNDS)[0]
    m1b = lax.gather(mask_v[pl.ds(L, L)], bvec, dnums, (1,),
                     mode=lax.GatherScatterMode.PROMISE_IN_BOUNDS)[0]

    ms = jnp.where(m0b != 0, 0, 1)
    me = jnp.where(m1b != 0, 1, ms)

    @pl.when(jnp.logical_or(ms != 0, me != 1))
    def _():
        f0 = pltpu.async_copy(
            emb_hbm.at[b, ms, pl.ds(c0, HALF)], d_v.at[0], sem0)
        f1 = pltpu.async_copy(
            emb_hbm.at[b, me, pl.ds(c0, HALF)], d_v.at[1], sem1)
        f0.wait()
        f1.wait()

    for k in range(0, HALF, L):
        d_v[0, pl.ds(k, L)] = (
            d_v[0, pl.ds(k, L)] + d_v[1, pl.ds(k, L)]) * 0.5

    pltpu.sync_copy(d_v.at[0], out_hbm.at[b, pl.ds(c0, HALF)])


def kernel(sequence_embeddings, special_tokens_mask):
    mask_rep = special_tokens_mask.T.reshape(-1)
    mesh = plsc.VectorSubcoreMesh(core_axis_name="c", subcore_axis_name="s")
    return pl.kernel(
        _body,
        out_type=jax.ShapeDtypeStruct((B, D), jnp.float32),
        mesh=mesh,
        scratch_types=[
            pltpu.VMEM((2 * L,), jnp.int32),
            pltpu.VMEM((2, HALF), jnp.float32),
            pltpu.SemaphoreType.DMA,
            pltpu.SemaphoreType.DMA,
        ],
    )(sequence_embeddings, mask_rep)
